# baseline (speedup 1.0000x reference)
import functools

import jax
import jax.numpy as jnp
import numpy as np
from jax.experimental import pallas as pl
from jax.experimental.pallas import tpu as pltpu

EPS = 1e-20


def _make_copy(neigh_hbm, scratch, sems, step, slot, kk, blk, d):
    return pltpu.make_async_copy(
        neigh_hbm.at[pl.ds(step * blk, blk), kk, :],
        scratch.at[slot, :, pl.ds(kk * d, d)],
        sems.at[slot, kk])


def _fused_block(k_count, blk, temp_ref, topk_ref, self_ref, neigh_hbm,
                 gu_ref, attw_ref, nw_ref, sw_ref, g_ref, gt_ref, h_ref,
                 out_ref, ew_ref, scratch, sems):
    K = int(k_count)
    D = self_ref.shape[1]
    i = pl.program_id(0)
    nsteps = pl.num_programs(0)

    @pl.when(i == 0)
    def _():
        for kk in range(K):
            _make_copy(neigh_hbm, scratch, sems, i, 0, kk, blk, D).start()

    @pl.when(i + 1 < nsteps)
    def _():
        for kk in range(K):
            _make_copy(neigh_hbm, scratch, sems, i + 1, (i + 1) % 2, kk,
                       blk, D).start()

    s = self_ref[...]
    q = jnp.dot(s, attw_ref[...], preferred_element_type=jnp.float32)
    g = -jnp.log(-jnp.log(gu_ref[...] + EPS) + EPS)

    slot = i % 2
    for kk in range(K):
        _make_copy(neigh_hbm, scratch, sems, i, slot, kk, blk, D).wait()

    nb = scratch[slot]
    qt = jnp.concatenate([q] * K, axis=1)
    ew = jnp.dot(nb * qt, g_ref[...],
                 preferred_element_type=jnp.float32)
    ew_ref[...] = ew

    inv_t = 1.0 / temp_ref[0]
    top_k = topk_ref[0].astype(jnp.float32)
    v = (ew + g) * inv_t
    v = v - jnp.max(v, axis=-1, keepdims=True)
    e = jnp.exp(v)
    mv = e / jnp.sum(e, axis=-1, keepdims=True)

    mvt = mv.T
    cntt = jnp.zeros_like(mvt)
    for kk in range(K):
        cntt = cntt + (mvt[kk:kk + 1, :] > mvt).astype(jnp.float32)
    maskt = (cntt < top_k).astype(jnp.float32)
    support = (mvt * maskt).T * (1.0 / k_count)

    st = jnp.dot(support, gt_ref[...],
                 preferred_element_type=jnp.float32)
    nm = jnp.dot(nb * st, h_ref[...],
                 preferred_element_type=jnp.float32)
    fn = jnp.dot(nm, nw_ref[...], preferred_element_type=jnp.float32)
    fs = jnp.dot(s, sw_ref[...], preferred_element_type=jnp.float32)
    out_ref[...] = jnp.maximum(fs + fn, 0.0)


def kernel(self_vecs, neigh_vecs, temperature, gumbel_u, att_weights,
           neigh_weights, self_weights, top_k):
    n, k, d = neigh_vecs.shape
    o = neigh_weights.shape[1]
    block = 1000
    grid = (n // block,)

    j = np.arange(k * d)
    gmat = jnp.asarray(j[:, None] // d == np.arange(k)[None, :],
                       dtype=jnp.float32)
    gmat_t = jnp.asarray(j[None, :] // d == np.arange(k)[:, None],
                         dtype=jnp.float32)
    hmat = jnp.asarray(j[:, None] % d == np.arange(d)[None, :],
                       dtype=jnp.float32)

    body = functools.partial(_fused_block, float(k), block)
    out, ew = pl.pallas_call(
        body,
        grid=grid,
        in_specs=[
            pl.BlockSpec(memory_space=pltpu.SMEM),
            pl.BlockSpec(memory_space=pltpu.SMEM),
            pl.BlockSpec((block, d), lambda i: (i, 0)),
            pl.BlockSpec(memory_space=pl.ANY),
            pl.BlockSpec((block, k), lambda i: (i, 0)),
            pl.BlockSpec((d, d), lambda i: (0, 0)),
            pl.BlockSpec((d, o), lambda i: (0, 0)),
            pl.BlockSpec((d, o), lambda i: (0, 0)),
            pl.BlockSpec((k * d, k), lambda i: (0, 0)),
            pl.BlockSpec((k, k * d), lambda i: (0, 0)),
            pl.BlockSpec((k * d, d), lambda i: (0, 0)),
        ],
        out_specs=[
            pl.BlockSpec((block, o), lambda i: (i, 0)),
            pl.BlockSpec((block, k), lambda i: (i, 0)),
        ],
        out_shape=[
            jax.ShapeDtypeStruct((n, o), jnp.float32),
            jax.ShapeDtypeStruct((n, k), jnp.float32),
        ],
        scratch_shapes=[
            pltpu.VMEM((2, block, k * d), jnp.float32),
            pltpu.SemaphoreType.DMA((2, k)),
        ],
    )(temperature.reshape(1), jnp.asarray(top_k).reshape(1),
      self_vecs, neigh_vecs, gumbel_u,
      att_weights, neigh_weights, self_weights, gmat, gmat_t, hmat)
    return (out, ew)


# speedup vs baseline: 1.0230x; 1.0230x over previous
import functools

import jax
import jax.numpy as jnp
import numpy as np
from jax.experimental import pallas as pl
from jax.experimental.pallas import tpu as pltpu

EPS = 1e-20


def _make_copy(neigh_hbm, scratch, sems, step, slot, kk, blk, d):
    return pltpu.make_async_copy(
        neigh_hbm.at[pl.ds(step * blk, blk), kk, :],
        scratch.at[slot, :, pl.ds(kk * d, d)],
        sems.at[slot, kk])


def _fused_block(k_count, blk, temp_ref, topk_ref, self_ref, neigh_hbm,
                 gu_ref, attw_ref, nw_ref, sw_ref, g_ref, gt_ref, h_ref,
                 out_ref, ew_ref, scratch, sems):
    K = int(k_count)
    D = self_ref.shape[1]
    i = pl.program_id(0)
    nsteps = pl.num_programs(0)

    @pl.when(i == 0)
    def _():
        for kk in range(K):
            _make_copy(neigh_hbm, scratch, sems, i, 0, kk, blk, D).start()

    @pl.when(i + 1 < nsteps)
    def _():
        for kk in range(K):
            _make_copy(neigh_hbm, scratch, sems, i + 1, (i + 1) % 2, kk,
                       blk, D).start()

    slot = i % 2
    for kk in range(K):
        _make_copy(neigh_hbm, scratch, sems, i, slot, kk, blk, D).wait()

    s = self_ref[...]
    q = jnp.dot(s, attw_ref[...], preferred_element_type=jnp.float32)

    nb = scratch[slot]
    qt = jnp.concatenate([q] * K, axis=1)
    ew = jnp.dot(nb * qt, g_ref[...],
                 preferred_element_type=jnp.float32)
    ew_ref[...] = ew

    inv_t = 1.0 / temp_ref[0]
    top_k = topk_ref[0].astype(jnp.float32)
    g = -jnp.log(-jnp.log(gu_ref[...] + EPS) + EPS)
    v = (ew + g) * inv_t
    v = v - jnp.max(v, axis=-1, keepdims=True)
    e = jnp.exp(v)
    mv = e / jnp.sum(e, axis=-1, keepdims=True)

    mvt = mv.T
    cntt = jnp.zeros_like(mvt)
    for kk in range(K):
        cntt = cntt + (mvt[kk:kk + 1, :] > mvt).astype(jnp.float32)
    maskt = (cntt < top_k).astype(jnp.float32)
    support = (mvt * maskt).T * (1.0 / k_count)

    st = jnp.dot(support, gt_ref[...],
                 preferred_element_type=jnp.float32)
    nm = jnp.dot(nb * st, h_ref[...],
                 preferred_element_type=jnp.float32)
    fn = jnp.dot(nm, nw_ref[...], preferred_element_type=jnp.float32)
    fs = jnp.dot(s, sw_ref[...], preferred_element_type=jnp.float32)
    out_ref[...] = jnp.maximum(fs + fn, 0.0)


def kernel(self_vecs, neigh_vecs, temperature, gumbel_u, att_weights,
           neigh_weights, self_weights, top_k):
    n, k, d = neigh_vecs.shape
    o = neigh_weights.shape[1]
    block = 1000
    grid = (n // block,)

    j = np.arange(k * d)
    gmat = jnp.asarray(j[:, None] // d == np.arange(k)[None, :],
                       dtype=jnp.float32)
    gmat_t = jnp.asarray(j[None, :] // d == np.arange(k)[:, None],
                         dtype=jnp.float32)
    hmat = jnp.asarray(j[:, None] % d == np.arange(d)[None, :],
                       dtype=jnp.float32)

    body = functools.partial(_fused_block, float(k), block)
    out, ew = pl.pallas_call(
        body,
        grid=grid,
        in_specs=[
            pl.BlockSpec(memory_space=pltpu.SMEM),
            pl.BlockSpec(memory_space=pltpu.SMEM),
            pl.BlockSpec((block, d), lambda i: (i, 0)),
            pl.BlockSpec(memory_space=pl.ANY),
            pl.BlockSpec((block, k), lambda i: (i, 0)),
            pl.BlockSpec((d, d), lambda i: (0, 0)),
            pl.BlockSpec((d, o), lambda i: (0, 0)),
            pl.BlockSpec((d, o), lambda i: (0, 0)),
            pl.BlockSpec((k * d, k), lambda i: (0, 0)),
            pl.BlockSpec((k, k * d), lambda i: (0, 0)),
            pl.BlockSpec((k * d, d), lambda i: (0, 0)),
        ],
        out_specs=[
            pl.BlockSpec((block, o), lambda i: (i, 0)),
            pl.BlockSpec((block, k), lambda i: (i, 0)),
        ],
        out_shape=[
            jax.ShapeDtypeStruct((n, o), jnp.float32),
            jax.ShapeDtypeStruct((n, k), jnp.float32),
        ],
        scratch_shapes=[
            pltpu.VMEM((2, block, k * d), jnp.float32),
            pltpu.SemaphoreType.DMA((2, k)),
        ],
    )(temperature.reshape(1), jnp.asarray(top_k).reshape(1),
      self_vecs, neigh_vecs, gumbel_u,
      att_weights, neigh_weights, self_weights, gmat, gmat_t, hmat)
    return (out, ew)
